# baseline (speedup 1.0000x reference)
import functools

import jax
import jax.numpy as jnp
from jax import lax
from jax.experimental import pallas as pl
from jax.experimental.pallas import tpu as pltpu
from jax.experimental.pallas import tpu_sc as plsc

_DIM = 32
_NC = 2
_NS = 16
_NW = _NC * _NS
_LANES = 16


@functools.lru_cache(maxsize=None)
def _build_gather(batch, hist):
    assert batch % (_NW * 128) == 0
    bpw = batch // _NW
    nbt = bpw // 128
    mesh = plsc.VectorSubcoreMesh(core_axis_name="c", subcore_axis_name="s")

    tile_words = (_DIM // 8) * bpw * 8
    dt_stride = (batch // 128) * 8 * 128
    chunk = bpw * 8

    @functools.partial(
        pl.kernel,
        out_type=jax.ShapeDtypeStruct(
            (hist, (_DIM // 8) * dt_stride), jnp.float32),
        mesh=mesh,
        compiler_params=pltpu.CompilerParams(
            use_tc_tiling_on_sc=False, needs_layout_passes=False),
        scratch_types=[
            pltpu.VMEM((hist, bpw), jnp.int32),
            pltpu.VMEM((bpw, _DIM), jnp.float32),
            pltpu.VMEM((bpw, _DIM), jnp.float32),
            pltpu.VMEM((tile_words,), jnp.float32),
            pltpu.VMEM((tile_words,), jnp.float32),
            pltpu.SemaphoreType.DMA,
            pltpu.SemaphoreType.DMA,
            pltpu.SemaphoreType.DMA,
            pltpu.SemaphoreType.DMA,
        ],
    )
    def grab(idx_hbm, table_hbm, out_hbm, idx_v, rows0, rows1,
             tile0, tile1, gsem0, gsem1, osem0, osem1):
        wid = lax.axis_index("s") * _NC + lax.axis_index("c")
        b0 = wid * bpw
        pltpu.sync_copy(idx_hbm.at[:, pl.ds(b0, bpw)], idx_v)
        rows = (rows0, rows1)
        tile = (tile0, tile1)
        gsem = (gsem0, gsem1)
        osem = (osem0, osem1)
        lane = lax.iota(jnp.int32, _LANES)
        dpat = tuple(
            ((d0 + lane) // 8) * (nbt * 1024) + ((d0 + lane) % 8) * 128
            for d0 in (0, _LANES)
        )
        cpat = tuple(d0 + lane for d0 in (0, _LANES))

        def fire_gather(h, b):
            pltpu.async_copy(table_hbm.at[idx_v.at[h]], rows[b], gsem[b])

        def drain_gather(b):
            pltpu.make_async_copy(
                table_hbm.at[pl.ds(0, bpw)], rows[b], gsem[b]).wait()

        def fire_out(h, b):
            for dt in range(_DIM // 8):
                pltpu.async_copy(
                    tile[b].at[pl.ds(dt * chunk, chunk)],
                    out_hbm.at[h, pl.ds(dt * dt_stride + wid * chunk, chunk)],
                    osem[b])

        def drain_out(b):
            pltpu.make_async_copy(
                tile[b], out_hbm.at[0, pl.ds(0, tile_words)], osem[b]).wait()

        def transpose(b):
            for j in range(_LANES):
                rot = (lane + j) % _LANES
                dsts = tuple(d + rot for d in dpat)

                @plsc.parallel_loop(0, bpw, step=_LANES, unroll=4)
                def _blk(r0):
                    c = (r0 // 128) * 1024 + (r0 % 128)
                    row_ids = r0 + rot
                    for half in (0, 1):
                        vals = plsc.load_gather(
                            rows[b], [row_ids, cpat[half]])
                        plsc.store_scatter(tile[b], [dsts[half] + c], vals)

        fire_gather(0, 0)
        fire_gather(1, 1)

        @pl.loop(0, hist - 2, step=2)
        def _steady(g):
            for b in (0, 1):
                h = g + b
                drain_gather(b)

                @pl.when(h >= 2)
                def _():
                    drain_out(b)

                transpose(b)
                fire_gather(h + 2, b)
                fire_out(h, b)

        for b in (0, 1):
            h = hist - 2 + b
            drain_gather(b)
            drain_out(b)
            transpose(b)
            fire_out(h, b)
        for b in (0, 1):
            drain_out(b)

    return grab


def kernel(speaker_labeles, table):
    batch, hist = speaker_labeles.shape
    idx_t = speaker_labeles.astype(jnp.int32).T
    out2 = _build_gather(batch, hist)(idx_t, table)
    out5 = out2.reshape(hist, _DIM // 8, batch // 128, 8, 128)
    return jnp.transpose(out5, (2, 4, 0, 1, 3)).reshape(batch, hist, _DIM)


# speedup vs baseline: 1.0997x; 1.0997x over previous
import functools

import jax
import jax.numpy as jnp
from jax import lax
from jax.experimental import pallas as pl
from jax.experimental.pallas import tpu as pltpu
from jax.experimental.pallas import tpu_sc as plsc

_DIM = 32
_NC = 2
_NS = 16
_NW = _NC * _NS
_LANES = 16


@functools.lru_cache(maxsize=None)
def _build_gather(batch, hist):
    assert batch % (_NW * 128) == 0
    bpw = batch // _NW
    nbt = bpw // 128
    mesh = plsc.VectorSubcoreMesh(core_axis_name="c", subcore_axis_name="s")

    tile_words = (_DIM // 8) * bpw * 8
    dt_stride = (batch // 128) * 8 * 128
    chunk = bpw * 8

    @functools.partial(
        pl.kernel,
        out_type=jax.ShapeDtypeStruct(
            (hist, (_DIM // 8) * dt_stride), jnp.float32),
        mesh=mesh,
        compiler_params=pltpu.CompilerParams(
            use_tc_tiling_on_sc=False, needs_layout_passes=False),
        scratch_types=[
            pltpu.VMEM((hist, bpw), jnp.int32),
            pltpu.VMEM((bpw, _DIM), jnp.float32),
            pltpu.VMEM((bpw, _DIM), jnp.float32),
            pltpu.VMEM((bpw, _DIM), jnp.float32),
            pltpu.VMEM((tile_words,), jnp.float32),
            pltpu.VMEM((tile_words,), jnp.float32),
            pltpu.SemaphoreType.DMA,
            pltpu.SemaphoreType.DMA,
            pltpu.SemaphoreType.DMA,
            pltpu.SemaphoreType.DMA,
            pltpu.SemaphoreType.DMA,
        ],
    )
    def grab(idx_hbm, table_hbm, out_hbm, idx_v, rows0, rows1, rows2,
             tile0, tile1, gsem0, gsem1, gsem2, osem0, osem1):
        wid = lax.axis_index("s") * _NC + lax.axis_index("c")
        b0 = wid * bpw
        pltpu.sync_copy(idx_hbm.at[:, pl.ds(b0, bpw)], idx_v)
        rows = (rows0, rows1, rows2)
        tile = (tile0, tile1)
        gsem = (gsem0, gsem1, gsem2)
        osem = (osem0, osem1)
        lane = lax.iota(jnp.int32, _LANES)
        dpat = tuple(
            ((d0 + lane) // 8) * (nbt * 1024) + ((d0 + lane) % 8) * 128
            for d0 in (0, _LANES)
        )
        cpat = tuple(d0 + lane for d0 in (0, _LANES))

        def fire_gather(h, b):
            pltpu.async_copy(table_hbm.at[idx_v.at[h]], rows[b], gsem[b])

        def drain_gather(b):
            pltpu.make_async_copy(
                table_hbm.at[pl.ds(0, bpw)], rows[b], gsem[b]).wait()

        def fire_out(h, b):
            for dt in range(_DIM // 8):
                pltpu.async_copy(
                    tile[b].at[pl.ds(dt * chunk, chunk)],
                    out_hbm.at[h, pl.ds(dt * dt_stride + wid * chunk, chunk)],
                    osem[b])

        def drain_out(b):
            pltpu.make_async_copy(
                tile[b], out_hbm.at[0, pl.ds(0, tile_words)], osem[b]).wait()

        def transpose(rb, tb):
            for j in range(_LANES):
                rot = (lane + j) % _LANES
                dsts = tuple(d + rot for d in dpat)

                @plsc.parallel_loop(0, bpw, step=_LANES, unroll=2)
                def _blk(r0):
                    c = (r0 // 128) * 1024 + (r0 % 128)
                    row_ids = r0 + rot
                    for half in (0, 1):
                        vals = plsc.load_gather(
                            rows[rb], [row_ids, cpat[half]])
                        plsc.store_scatter(tile[tb], [dsts[half] + c], vals)

        fire_gather(0, 0)
        fire_gather(1, 1)
        assert (hist - 2) % 6 == 0

        @pl.loop(0, hist - 2, step=6)
        def _steady(g):
            for i in range(6):
                h = g + i
                rb, tb = i % 3, i % 2
                drain_gather(rb)
                fire_gather(h + 2, (i + 2) % 3)

                @pl.when(h >= 2)
                def _():
                    drain_out(tb)

                transpose(rb, tb)
                fire_out(h, tb)

        for i in range(2):
            h = hist - 2 + i
            rb, tb = h % 3, h % 2
            drain_gather(rb)
            drain_out(tb)
            transpose(rb, tb)
            fire_out(h, tb)
        for tb in (0, 1):
            drain_out(tb)

    return grab


def kernel(speaker_labeles, table):
    batch, hist = speaker_labeles.shape
    idx_t = speaker_labeles.astype(jnp.int32).T
    out2 = _build_gather(batch, hist)(idx_t, table)
    out5 = out2.reshape(hist, _DIM // 8, batch // 128, 8, 128)
    return jnp.transpose(out5, (2, 4, 0, 1, 3)).reshape(batch, hist, _DIM)
